# baseline (speedup 1.0000x reference)
import functools

import jax
import jax.numpy as jnp
from jax import lax
from jax.experimental import pallas as pl
from jax.experimental.pallas import tpu as pltpu
from jax.experimental.pallas import tpu_sc as plsc

N, D = 16384, 2048
NC, NS = 2, 16
NW = NC * NS
ROWS_W = N // NW
CHUNK = 8
NCHUNK = ROWS_W // CHUNK
GPC = CHUNK // 4
NBUF = 6
AHEAD = 4


def _sc_body(x_hbm, o_hbm, *refs):
    bufs = refs[:NBUF]
    sins = refs[NBUF:2 * NBUF]
    souts = refs[2 * NBUF:3 * NBUF]

    wid = lax.axis_index("s") * NC + lax.axis_index("c")
    base = wid * ROWS_W

    zero = jnp.zeros((16,), jnp.float32)

    def gather(k, b):
        return pltpu.make_async_copy(
            x_hbm.at[pl.ds(base + k * CHUNK, CHUNK)], bufs[b], sins[b]
        )

    def write(k, b):
        return pltpu.make_async_copy(
            bufs[b], o_hbm.at[pl.ds(base + k * CHUNK, CHUNK)], souts[b]
        )

    for k0 in range(AHEAD):
        gather(k0, k0).start()

    def step(k, c):
        for b in range(NBUF):

            @pl.when(k % NBUF == b)
            def _():
                gather(k, b).wait()

                def zrow(j, cc):
                    for g in range(GPC):
                        for u in range(4):
                            bufs[b][4 * g, pl.ds((4 * j + u) * 16, 16)] = zero
                    return cc

                lax.fori_loop(0, D // 64, zrow, 0)
                write(k, b).start()

                @pl.when(k + AHEAD < NCHUNK)
                def _():
                    bn = (b + AHEAD) % NBUF

                    @pl.when(k >= NBUF - AHEAD)
                    def _():
                        write(k + AHEAD - NBUF, bn).wait()

                    gather(k + AHEAD, bn).start()

        return c

    lax.fori_loop(0, NCHUNK, step, 0)

    for k in range(max(0, NCHUNK - NBUF), NCHUNK):
        write(k, k % NBUF).wait()


_sc_kernel = functools.partial(
    pl.kernel,
    mesh=plsc.VectorSubcoreMesh(core_axis_name="c", subcore_axis_name="s"),
    out_type=jax.ShapeDtypeStruct((N, D), jnp.float32),
    scratch_types=(
        [pltpu.VMEM((CHUNK, D), jnp.float32)] * NBUF
        + [pltpu.SemaphoreType.DMA] * (2 * NBUF)
    ),
)(_sc_body)


def kernel(x):
    return _sc_kernel(x)


# speedup vs baseline: 1.0050x; 1.0050x over previous
import functools

import jax
import jax.numpy as jnp
from jax import lax
from jax.experimental import pallas as pl
from jax.experimental.pallas import tpu as pltpu
from jax.experimental.pallas import tpu_sc as plsc

N, D = 16384, 2048
NC, NS = 2, 16
NW = NC * NS
ROWS_W = N // NW
CHUNK = 8
NCHUNK = ROWS_W // CHUNK
GPC = CHUNK // 4
NBUF = 6
AHEAD = 4


def _sc_body(x_hbm, o_hbm, *refs):
    bufs = refs[:NBUF]
    sins = refs[NBUF:2 * NBUF]
    souts = refs[2 * NBUF:3 * NBUF]

    wid = lax.axis_index("c") * NS + lax.axis_index("s")
    base = wid * ROWS_W

    zero = jnp.zeros((16,), jnp.float32)

    def gather(k, b):
        return pltpu.make_async_copy(
            x_hbm.at[pl.ds(base + k * CHUNK, CHUNK)], bufs[b], sins[b]
        )

    def write(k, b):
        return pltpu.make_async_copy(
            bufs[b], o_hbm.at[pl.ds(base + k * CHUNK, CHUNK)], souts[b]
        )

    for k0 in range(AHEAD):
        gather(k0, k0).start()

    def step(k, c):
        for b in range(NBUF):

            @pl.when(k % NBUF == b)
            def _():
                gather(k, b).wait()

                def zrow(j, cc):
                    for g in range(GPC):
                        for u in range(4):
                            bufs[b][4 * g, pl.ds((4 * j + u) * 16, 16)] = zero
                    return cc

                lax.fori_loop(0, D // 64, zrow, 0)
                write(k, b).start()

                @pl.when(k + AHEAD < NCHUNK)
                def _():
                    bn = (b + AHEAD) % NBUF

                    @pl.when(k >= NBUF - AHEAD)
                    def _():
                        write(k + AHEAD - NBUF, bn).wait()

                    gather(k + AHEAD, bn).start()

        return c

    lax.fori_loop(0, NCHUNK, step, 0)

    for k in range(max(0, NCHUNK - NBUF), NCHUNK):
        write(k, k % NBUF).wait()


_sc_kernel = functools.partial(
    pl.kernel,
    mesh=plsc.VectorSubcoreMesh(core_axis_name="c", subcore_axis_name="s"),
    out_type=jax.ShapeDtypeStruct((N, D), jnp.float32),
    scratch_types=(
        [pltpu.VMEM((CHUNK, D), jnp.float32)] * NBUF
        + [pltpu.SemaphoreType.DMA] * (2 * NBUF)
    ),
)(_sc_body)


def kernel(x):
    return _sc_kernel(x)
